# baseline (speedup 1.0000x reference)
import functools

import jax
import jax.numpy as jnp
from jax import lax
from jax.experimental import pallas as pl
from jax.experimental.pallas import tpu as pltpu
from jax.experimental.pallas import tpu_sc as plsc

NC, NS, L = 2, 16, 16
NW = NC * NS
B, H, E, O = 4096, 200, 64, 128
RPW = B // NW
HALF = H // 2
G = E // L

_MESH = plsc.VectorSubcoreMesh(core_axis_name="c", subcore_axis_name="s",
                               num_cores=NC, num_subcores=NS)


@functools.partial(
    pl.kernel,
    out_type=jax.ShapeDtypeStruct((B, E), jnp.float32),
    mesh=_MESH,
    scratch_types=[
        pltpu.VMEM((RPW, 2, HALF), jnp.int32),
        pltpu.VMEM((2, HALF, E), jnp.float32),
        pltpu.VMEM((RPW, E), jnp.float32),
        pltpu.SemaphoreType.DMA,
    ],
    compiler_params=pltpu.CompilerParams(use_tc_tiling_on_sc=False),
)
def _pool(tok_hbm, emb_hbm, out_hbm, idx_v, buf_v, acc_v, sem):
    wid = lax.axis_index("s") * NC + lax.axis_index("c")
    base = wid * RPW
    pltpu.sync_copy(tok_hbm.at[wid], idx_v)

    def row_body(r, _):
        pltpu.async_copy(emb_hbm.at[idx_v.at[r, 0]], buf_v.at[0], sem).wait()
        pltpu.async_copy(emb_hbm.at[idx_v.at[r, 1]], buf_v.at[1], sem).wait()

        def sum_body(i, accs):
            new = []
            for g in range(G):
                a = accs[g]
                a = a + buf_v[0, i, pl.ds(g * L, L)]
                a = a + buf_v[1, i, pl.ds(g * L, L)]
                new.append(a)
            return tuple(new)

        zeros = tuple(jnp.zeros((L,), jnp.float32) for _ in range(G))
        accs = lax.fori_loop(0, HALF, sum_body, zeros)
        for g in range(G):
            acc_v[r, pl.ds(g * L, L)] = accs[g]
        return 0

    lax.fori_loop(0, RPW, row_body, 0)
    pltpu.sync_copy(acc_v, out_hbm.at[pl.ds(base, RPW)])


def _proj_body(p_ref, w_ref, b_ref, o_ref):
    pooled = p_ref[...] * jnp.float32(1.0 / H)
    o_ref[...] = jnp.dot(pooled, w_ref[...],
                         preferred_element_type=jnp.float32) + b_ref[...]


def _proj(pooled, W, b2):
    return pl.pallas_call(
        _proj_body,
        out_shape=jax.ShapeDtypeStruct((B, O), jnp.float32),
    )(pooled, W, b2)


def kernel(token_ids, embedding, W, b):
    tok = token_ids.reshape(NW, RPW, 2, HALF)
    pooled = _pool(tok, embedding)
    return _proj(pooled, W, b.reshape(1, O))


# speedup vs baseline: 1.1714x; 1.1714x over previous
import functools

import jax
import jax.numpy as jnp
from jax import lax
from jax.experimental import pallas as pl
from jax.experimental.pallas import tpu as pltpu
from jax.experimental.pallas import tpu_sc as plsc

NC, NS, L = 2, 16, 16
NW = NC * NS
B, H, E, O = 4096, 200, 64, 128
RPW = B // NW
HALF = H // 2
G = E // L

_MESH = plsc.VectorSubcoreMesh(core_axis_name="c", subcore_axis_name="s",
                               num_cores=NC, num_subcores=NS)


@functools.partial(
    pl.kernel,
    out_type=jax.ShapeDtypeStruct((B, E), jnp.float32),
    mesh=_MESH,
    scratch_types=[
        pltpu.VMEM((RPW, 2, HALF), jnp.int32),
        pltpu.VMEM((2, 2, HALF, E), jnp.float32),
        pltpu.VMEM((RPW, E), jnp.float32),
        pltpu.SemaphoreType.DMA,
        pltpu.SemaphoreType.DMA,
    ],
    compiler_params=pltpu.CompilerParams(use_tc_tiling_on_sc=False),
)
def _pool(tok_hbm, emb_hbm, out_hbm, idx_v, buf_v, acc_v, sem0, sem1):
    wid = lax.axis_index("s") * NC + lax.axis_index("c")
    base = wid * RPW
    pltpu.sync_copy(tok_hbm.at[wid], idx_v)

    sems = (sem0, sem1)

    def start(r, slot):
        pltpu.async_copy(emb_hbm.at[idx_v.at[r, 0]], buf_v.at[slot, 0],
                         sems[slot])
        pltpu.async_copy(emb_hbm.at[idx_v.at[r, 1]], buf_v.at[slot, 1],
                         sems[slot])

    def drain(slot):
        pltpu.make_async_copy(emb_hbm.at[idx_v.at[0, 0]],
                              buf_v.at[slot, 0], sems[slot]).wait()
        pltpu.make_async_copy(emb_hbm.at[idx_v.at[0, 1]],
                              buf_v.at[slot, 1], sems[slot]).wait()

    def consume(r, slot):
        def sum_body(i, accs):
            new = []
            for g in range(G):
                a = accs[g]
                a = a + buf_v[slot, 0, i, pl.ds(g * L, L)]
                a = a + buf_v[slot, 1, i, pl.ds(g * L, L)]
                new.append(a)
            return tuple(new)

        zeros = tuple(jnp.zeros((L,), jnp.float32) for _ in range(G))
        accs = lax.fori_loop(0, HALF, sum_body, zeros)
        for g in range(G):
            acc_v[r, pl.ds(g * L, L)] = accs[g]

    start(0, 0)

    def pair_body(rr, _):
        r0 = 2 * rr
        drain(0)
        start(r0 + 1, 1)
        consume(r0, 0)
        drain(1)

        @pl.when(rr < RPW // 2 - 1)
        def _():
            start(r0 + 2, 0)

        consume(r0 + 1, 1)
        return 0

    lax.fori_loop(0, RPW // 2, pair_body, 0)
    pltpu.sync_copy(acc_v, out_hbm.at[pl.ds(base, RPW)])


def _proj_body(p_ref, w_ref, b_ref, o_ref):
    pooled = p_ref[...] * jnp.float32(1.0 / H)
    o_ref[...] = jnp.dot(pooled, w_ref[...],
                         preferred_element_type=jnp.float32) + b_ref[...]


def _proj(pooled, W, b2):
    return pl.pallas_call(
        _proj_body,
        out_shape=jax.ShapeDtypeStruct((B, O), jnp.float32),
    )(pooled, W, b2)


def kernel(token_ids, embedding, W, b):
    tok = token_ids.reshape(NW, RPW, 2, HALF)
    pooled = _pool(tok, embedding)
    return _proj(pooled, W, b.reshape(1, O))
